# Initial kernel scaffold; baseline (speedup 1.0000x reference)
import jax
import jax.numpy as jnp
from jax.experimental import pallas as pl


def kernel(pred, target):
    raise NotImplementedError("write your pallas kernel here")


# speedup vs baseline: 18.3501x; 18.3501x over previous
import functools

import jax
import jax.numpy as jnp
from jax import lax
from jax.experimental import pallas as pl
from jax.experimental.pallas import tpu as pltpu
from jax.experimental.pallas import tpu_sc as plsc

_B = 8
_N = 512 * 512
_KK = 5242
_L = 16
_TPE = 4
_EPC = 4
_CHUNK = _N // _TPE
_HB = 2048
_BIG = 2**30

_LEVELS = ((20, 0), (9, -(1 << 20)), (0, -(1 << 9)))


def _bce_body(p_ref, t_ref, o_ref):
    p = p_ref[...]
    t = t_ref[...]
    lp = jnp.maximum(jnp.log(p), -100.0)
    l1p = jnp.maximum(jnp.log(1.0 - p), -100.0)
    o_ref[...] = -(t * lp + (1.0 - t) * l1p)


def _bce(pred, target):
    return pl.pallas_call(
        _bce_body,
        out_shape=jax.ShapeDtypeStruct((_B, 512, 512), jnp.float32),
        grid=(_B,),
        in_specs=[
            pl.BlockSpec((1, 512, 512), lambda i: (i, 0, 0)),
            pl.BlockSpec((1, 512, 512), lambda i: (i, 0, 0)),
        ],
        out_specs=pl.BlockSpec((1, 512, 512), lambda i: (i, 0, 0)),
    )(pred, target)


_sc_mesh = plsc.VectorSubcoreMesh(core_axis_name="c", subcore_axis_name="s")


@functools.partial(
    pl.kernel,
    out_type=jax.ShapeDtypeStruct((_B, _TPE, _L), jnp.float32),
    mesh=_sc_mesh,
    compiler_params=pltpu.CompilerParams(needs_layout_passes=False),
    scratch_types=[
        pltpu.VMEM((_CHUNK,), jnp.float32),
        pltpu.VMEM((_HB,), jnp.int32),
        pltpu.VMEM((_TPE, _HB), jnp.int32),
        pltpu.VMEM((_L,), jnp.float32),
        pltpu.VMEM_SHARED((16, _HB), jnp.int32),
    ],
)
def _select(loss_hbm, out_hbm, loss_v, hist_v, hist4_v, acc_v, sh_hist):
    c = lax.axis_index("c")
    s = lax.axis_index("s")
    ex = c * _EPC + s // _TPE
    q = s % _TPE
    base = (s // _TPE) * _TPE
    off = pl.multiple_of(ex * _N + q * _CHUNK, _CHUNK)
    pltpu.sync_copy(loss_hbm.at[pl.ds(off, _CHUNK)], loss_v)

    iota = lax.iota(jnp.int32, _L)
    ones_i = jnp.ones((_L,), jnp.int32)
    zeros_i = jnp.zeros((_L,), jnp.int32)

    prefix = jnp.int32(0)
    r = jnp.int32(_KK)
    n = jnp.int32(_N)

    for shift, pmask in _LEVELS:
        def zb(i, _):
            hist_v[pl.ds(i * _L, _L)] = zeros_i
            return 0
        lax.fori_loop(0, _HB // _L, zb, 0)

        pv = jnp.full((_L,), prefix & pmask, jnp.int32)
        def hb(i, _):
            x = loss_v[pl.ds(i * _L, _L)]
            bits = plsc.bitcast(x, jnp.int32)
            bucket = (bits >> shift) & (_HB - 1)
            if pmask == 0:
                plsc.addupdate_scatter(hist_v, [bucket], ones_i)
            else:
                m = (bits & pmask) == pv
                plsc.addupdate_scatter(hist_v, [bucket], ones_i, mask=m)
            return 0
        lax.fori_loop(0, _CHUNK // _L, hb, 0)

        pltpu.sync_copy(hist_v, sh_hist.at[s])
        plsc.subcore_barrier()
        pltpu.sync_copy(sh_hist.at[pl.ds(base, _TPE)], hist4_v)
        plsc.subcore_barrier()

        thresh = n - r
        def cb(i, carry2):
            cum, bstar, cstar, cbelow = carry2
            h = (hist4_v[0, pl.ds(i * _L, _L)]
                 + hist4_v[1, pl.ds(i * _L, _L)]
                 + hist4_v[2, pl.ds(i * _L, _L)]
                 + hist4_v[3, pl.ds(i * _L, _L)])
            cc = plsc.cumsum(h) + cum
            good = cc >= thresh
            big = jnp.int32(_BIG)
            bstar = jnp.minimum(bstar, jnp.min(jnp.where(good, iota + i * _L, big)))
            cstar = jnp.minimum(cstar, jnp.min(jnp.where(good, cc, big)))
            cbelow = jnp.maximum(cbelow, jnp.max(jnp.where(good, 0, cc)))
            return (jnp.max(cc), bstar, cstar, cbelow)
        _, bstar, cstar, cbelow = lax.fori_loop(
            0, _HB // _L, cb,
            (jnp.int32(0), jnp.int32(_BIG), jnp.int32(_BIG), jnp.int32(0)))

        prefix = prefix | (bstar << shift)
        r = r - (n - cstar)
        n = cstar - cbelow

    vv = plsc.bitcast(jnp.full((_L,), prefix, jnp.int32), jnp.float32)
    def fb(i, carry):
        sacc, cacc = carry
        x = loss_v[pl.ds(i * _L, _L)]
        m = x > vv
        return (sacc + jnp.where(m, x, 0.0), cacc + jnp.where(m, 1, 0))
    sacc, cacc = lax.fori_loop(
        0, _CHUNK // _L, fb, (jnp.zeros((_L,), jnp.float32), zeros_i))
    ssum = jnp.sum(sacc)
    scnt = jnp.sum(cacc).astype(jnp.float32)

    acc_v[...] = jnp.where(iota == 0, ssum, jnp.where(iota == 1, scnt, 0.0))
    pltpu.sync_copy(acc_v, out_hbm.at[ex, q])


def kernel(pred, target):
    p = pred.reshape(_B, 512, 512)
    t = target.reshape(_B, 512, 512)
    loss = _bce(p, t)
    acc = _select(loss.reshape(_B * _N))
    return acc[:, :, 0].sum(axis=1) / acc[:, :, 1].sum(axis=1)


# speedup vs baseline: 41.7549x; 2.2755x over previous
import functools

import jax
import jax.numpy as jnp
from jax import lax
from jax.experimental import pallas as pl
from jax.experimental.pallas import tpu as pltpu
from jax.experimental.pallas import tpu_sc as plsc

_B = 8
_N = 512 * 512
_KK = 5242
_L = 16
_TPE = 4
_EPC = 4
_CHUNK = _N // _TPE
_HB = 2048
_BIG = 2**30

_LEVELS = ((20, 0), (9, -(1 << 20)), (0, -(1 << 9)))


def _bce_body(p_ref, t_ref, o_ref):
    p = p_ref[...]
    t = t_ref[...]
    lp = jnp.maximum(jnp.log(p), -100.0)
    l1p = jnp.maximum(jnp.log(1.0 - p), -100.0)
    o_ref[...] = -(t * lp + (1.0 - t) * l1p)


def _bce(pred, target):
    return pl.pallas_call(
        _bce_body,
        out_shape=jax.ShapeDtypeStruct((_B, 512, 512), jnp.float32),
        grid=(_B,),
        in_specs=[
            pl.BlockSpec((1, 512, 512), lambda i: (i, 0, 0)),
            pl.BlockSpec((1, 512, 512), lambda i: (i, 0, 0)),
        ],
        out_specs=pl.BlockSpec((1, 512, 512), lambda i: (i, 0, 0)),
    )(pred, target)


_sc_mesh = plsc.VectorSubcoreMesh(core_axis_name="c", subcore_axis_name="s")


@functools.partial(
    pl.kernel,
    out_type=jax.ShapeDtypeStruct((_B, _TPE, _L), jnp.float32),
    mesh=_sc_mesh,
    compiler_params=pltpu.CompilerParams(needs_layout_passes=False),
    scratch_types=[
        pltpu.VMEM((_CHUNK,), jnp.float32),
        pltpu.VMEM((_HB,), jnp.int32),
        pltpu.VMEM((_TPE, _HB), jnp.int32),
        pltpu.VMEM((_L,), jnp.float32),
        pltpu.VMEM_SHARED((16, _HB), jnp.int32),
    ],
)
def _select(loss_hbm, out_hbm, loss_v, hist_v, hist4_v, acc_v, sh_hist):
    c = lax.axis_index("c")
    s = lax.axis_index("s")
    ex = c * _EPC + s // _TPE
    q = s % _TPE
    base = (s // _TPE) * _TPE
    off = pl.multiple_of(ex * _N + q * _CHUNK, _CHUNK)
    pltpu.sync_copy(loss_hbm.at[pl.ds(off, _CHUNK)], loss_v)

    iota = lax.iota(jnp.int32, _L)
    ones_i = jnp.ones((_L,), jnp.int32)
    zeros_i = jnp.zeros((_L,), jnp.int32)

    prefix = jnp.int32(0)
    r = jnp.int32(_KK)
    n = jnp.int32(_N)

    for shift, pmask in _LEVELS:
        @plsc.parallel_loop(0, _HB, _L, unroll=4)
        def _(i):
            hist_v[pl.ds(i, _L)] = zeros_i

        pv = jnp.full((_L,), prefix & pmask, jnp.int32)
        @plsc.parallel_loop(0, _CHUNK, _L, unroll=8)
        def _(i):
            x = loss_v[pl.ds(i, _L)]
            bits = plsc.bitcast(x, jnp.int32)
            bucket = (bits >> shift) & (_HB - 1)
            if pmask == 0:
                plsc.addupdate_scatter(hist_v, [bucket], ones_i)
            else:
                m = (bits & pmask) == pv
                plsc.addupdate_scatter(hist_v, [bucket], ones_i, mask=m)

        pltpu.sync_copy(hist_v, sh_hist.at[s])
        plsc.subcore_barrier()
        pltpu.sync_copy(sh_hist.at[pl.ds(base, _TPE)], hist4_v)
        plsc.subcore_barrier()

        thresh = n - r
        def cb(i, carry2):
            cum, bstar, cstar, cbelow = carry2
            h = (hist4_v[0, pl.ds(i * _L, _L)]
                 + hist4_v[1, pl.ds(i * _L, _L)]
                 + hist4_v[2, pl.ds(i * _L, _L)]
                 + hist4_v[3, pl.ds(i * _L, _L)])
            cc = plsc.cumsum(h) + cum
            good = cc >= thresh
            big = jnp.int32(_BIG)
            bstar = jnp.minimum(bstar, jnp.min(jnp.where(good, iota + i * _L, big)))
            cstar = jnp.minimum(cstar, jnp.min(jnp.where(good, cc, big)))
            cbelow = jnp.maximum(cbelow, jnp.max(jnp.where(good, 0, cc)))
            return (jnp.max(cc), bstar, cstar, cbelow)
        _, bstar, cstar, cbelow = lax.fori_loop(
            0, _HB // _L, cb,
            (jnp.int32(0), jnp.int32(_BIG), jnp.int32(_BIG), jnp.int32(0)))

        prefix = prefix | (bstar << shift)
        r = r - (n - cstar)
        n = cstar - cbelow

    vv = plsc.bitcast(jnp.full((_L,), prefix, jnp.int32), jnp.float32)
    @plsc.parallel_loop(0, _CHUNK, _L, unroll=8,
                        carry=(jnp.zeros((_L,), jnp.float32), zeros_i))
    def _fsums(i, carry):
        sacc, cacc = carry
        x = loss_v[pl.ds(i, _L)]
        m = x > vv
        return (sacc + jnp.where(m, x, 0.0), cacc + jnp.where(m, 1, 0))
    sacc, cacc = _fsums
    ssum = jnp.sum(sacc)
    scnt = jnp.sum(cacc).astype(jnp.float32)

    acc_v[...] = jnp.where(iota == 0, ssum, jnp.where(iota == 1, scnt, 0.0))
    pltpu.sync_copy(acc_v, out_hbm.at[ex, q])


def kernel(pred, target):
    p = pred.reshape(_B, 512, 512)
    t = target.reshape(_B, 512, 512)
    loss = _bce(p, t)
    acc = _select(loss.reshape(_B * _N))
    return acc[:, :, 0].sum(axis=1) / acc[:, :, 1].sum(axis=1)


# speedup vs baseline: 41.7723x; 1.0004x over previous
import functools

import jax
import jax.numpy as jnp
from jax import lax
from jax.experimental import pallas as pl
from jax.experimental.pallas import tpu as pltpu
from jax.experimental.pallas import tpu_sc as plsc

_B = 8
_N = 512 * 512
_KK = 5242
_L = 16
_TPE = 4
_EPC = 4
_CHUNK = _N // _TPE
_HB = 2048
_BIG = 2**30

_LEVELS = ((20, 0), (9, -(1 << 20)), (0, -(1 << 9)))


def _bce_body(p_ref, t_ref, o_ref):
    p = p_ref[...]
    t = t_ref[...]
    lp = jnp.maximum(jnp.log(p), -100.0)
    l1p = jnp.maximum(jnp.log(1.0 - p), -100.0)
    o_ref[...] = -(t * lp + (1.0 - t) * l1p)


def _bce(pred, target):
    return pl.pallas_call(
        _bce_body,
        out_shape=jax.ShapeDtypeStruct((_B, 512, 512), jnp.float32),
        grid=(_B,),
        in_specs=[
            pl.BlockSpec((1, 512, 512), lambda i: (i, 0, 0)),
            pl.BlockSpec((1, 512, 512), lambda i: (i, 0, 0)),
        ],
        out_specs=pl.BlockSpec((1, 512, 512), lambda i: (i, 0, 0)),
    )(pred, target)


_sc_mesh = plsc.VectorSubcoreMesh(core_axis_name="c", subcore_axis_name="s")


@functools.partial(
    pl.kernel,
    out_type=jax.ShapeDtypeStruct((_B, _TPE, _L), jnp.float32),
    mesh=_sc_mesh,
    compiler_params=pltpu.CompilerParams(needs_layout_passes=False),
    scratch_types=[
        pltpu.VMEM((_CHUNK,), jnp.float32),
        pltpu.VMEM((_HB,), jnp.int32),
        pltpu.VMEM((_TPE, _HB), jnp.int32),
        pltpu.VMEM((_L,), jnp.float32),
        pltpu.VMEM_SHARED((16, _HB), jnp.int32),
    ],
)
def _select(loss_hbm, out_hbm, loss_v, hist_v, hist4_v, acc_v, sh_hist):
    c = lax.axis_index("c")
    s = lax.axis_index("s")
    ex = c * _EPC + s // _TPE
    q = s % _TPE
    base = (s // _TPE) * _TPE
    off = pl.multiple_of(q * _CHUNK, _CHUNK)
    pltpu.sync_copy(loss_hbm.at[ex, pl.ds(off, _CHUNK)], loss_v)

    iota = lax.iota(jnp.int32, _L)
    ones_i = jnp.ones((_L,), jnp.int32)
    zeros_i = jnp.zeros((_L,), jnp.int32)

    prefix = jnp.int32(0)
    r = jnp.int32(_KK)
    n = jnp.int32(_N)

    for shift, pmask in _LEVELS:
        @plsc.parallel_loop(0, _HB, _L, unroll=4)
        def _(i):
            hist_v[pl.ds(i, _L)] = zeros_i

        pv = jnp.full((_L,), prefix & pmask, jnp.int32)
        @plsc.parallel_loop(0, _CHUNK, _L, unroll=8)
        def _(i):
            x = loss_v[pl.ds(i, _L)]
            bits = plsc.bitcast(x, jnp.int32)
            bucket = (bits >> shift) & (_HB - 1)
            if pmask == 0:
                plsc.addupdate_scatter(hist_v, [bucket], ones_i)
            else:
                m = (bits & pmask) == pv
                plsc.addupdate_scatter(hist_v, [bucket], ones_i, mask=m)

        pltpu.sync_copy(hist_v, sh_hist.at[s])
        plsc.subcore_barrier()
        pltpu.sync_copy(sh_hist.at[pl.ds(base, _TPE)], hist4_v)
        plsc.subcore_barrier()

        thresh = n - r
        def cb(i, carry2):
            cum, bstar, cstar, cbelow = carry2
            h = (hist4_v[0, pl.ds(i * _L, _L)]
                 + hist4_v[1, pl.ds(i * _L, _L)]
                 + hist4_v[2, pl.ds(i * _L, _L)]
                 + hist4_v[3, pl.ds(i * _L, _L)])
            cc = plsc.cumsum(h) + cum
            good = cc >= thresh
            big = jnp.int32(_BIG)
            bstar = jnp.minimum(bstar, jnp.min(jnp.where(good, iota + i * _L, big)))
            cstar = jnp.minimum(cstar, jnp.min(jnp.where(good, cc, big)))
            cbelow = jnp.maximum(cbelow, jnp.max(jnp.where(good, 0, cc)))
            return (jnp.max(cc), bstar, cstar, cbelow)
        _, bstar, cstar, cbelow = lax.fori_loop(
            0, _HB // _L, cb,
            (jnp.int32(0), jnp.int32(_BIG), jnp.int32(_BIG), jnp.int32(0)))

        prefix = prefix | (bstar << shift)
        r = r - (n - cstar)
        n = cstar - cbelow

    vv = plsc.bitcast(jnp.full((_L,), prefix, jnp.int32), jnp.float32)
    @plsc.parallel_loop(0, _CHUNK, _L, unroll=8,
                        carry=(jnp.zeros((_L,), jnp.float32), zeros_i))
    def _fsums(i, carry):
        sacc, cacc = carry
        x = loss_v[pl.ds(i, _L)]
        m = x > vv
        return (sacc + jnp.where(m, x, 0.0), cacc + jnp.where(m, 1, 0))
    sacc, cacc = _fsums
    ssum = jnp.sum(sacc)
    scnt = jnp.sum(cacc).astype(jnp.float32)

    acc_v[...] = jnp.where(iota == 0, ssum, jnp.where(iota == 1, scnt, 0.0))
    pltpu.sync_copy(acc_v, out_hbm.at[ex, q])


def kernel(pred, target):
    p = pred.reshape(_B, 512, 512)
    t = target.reshape(_B, 512, 512)
    loss = _bce(p, t)
    acc = _select(loss.reshape(_B, _N))
    return acc[:, :, 0].sum(axis=1) / acc[:, :, 1].sum(axis=1)
